# Initial kernel scaffold; baseline (speedup 1.0000x reference)
import jax
import jax.numpy as jnp
from jax.experimental import pallas as pl


def kernel(x, edge_index, batch, W1l, W1r, b1, g1, be1, W2l, W2r, b2, g2, be2, Wh, bh, g3, be3, Wf, bf):
    raise NotImplementedError("write your pallas kernel here")


# speedup vs baseline: 1.0011x; 1.0011x over previous
import functools

import jax
import jax.numpy as jnp
from jax import lax
from jax.experimental import pallas as pl
from jax.experimental.pallas import tpu as pltpu

N = 10000
E = 160000
TOP_K = 8


def _dense1_body(agg_ref, cnt_ref, x_ref, w1l_ref, w1r_ref, b1_ref, g1_ref,
                 be1_ref, h1_ref):
    cnt = cnt_ref[...]
    mean_agg = agg_ref[...] / jnp.maximum(cnt, 1.0)[:, None]
    p = (lax.dot_general(mean_agg, w1l_ref[...], (((1,), (1,)), ((), ())),
                         preferred_element_type=jnp.float32)
         + lax.dot_general(x_ref[...], w1r_ref[...], (((1,), (1,)), ((), ())),
                           preferred_element_type=jnp.float32)
         + b1_ref[...][None, :])
    mu = jnp.mean(p, axis=0, keepdims=True)
    var = jnp.mean(p * p, axis=0, keepdims=True) - mu * mu
    h = g1_ref[...][None, :] * (p - mu) / jnp.sqrt(var + 1e-5) + be1_ref[...][None, :]
    h1_ref[...] = jnp.maximum(h, 0.0)


def _dense2_body(agg_ref, cnt_ref, h1_ref, w2l_ref, w2r_ref, b2_ref, g2_ref,
                 be2_ref, wh_ref, bh_ref, g3_ref, be3_ref, wf_ref, bf_ref,
                 out_ref):
    cnt = cnt_ref[...]
    mean_agg = agg_ref[...] / jnp.maximum(cnt, 1.0)[:, None]
    p = (lax.dot_general(mean_agg, w2l_ref[...], (((1,), (1,)), ((), ())),
                         preferred_element_type=jnp.float32)
         + lax.dot_general(h1_ref[...], w2r_ref[...], (((1,), (1,)), ((), ())),
                           preferred_element_type=jnp.float32)
         + b2_ref[...][None, :])
    mu = jnp.mean(p, axis=0, keepdims=True)
    var = jnp.mean(p * p, axis=0, keepdims=True) - mu * mu
    h = g2_ref[...][None, :] * (p - mu) / jnp.sqrt(var + 1e-5) + be2_ref[...][None, :]
    h2 = jnp.maximum(h, 0.0)

    h3 = lax.dot_general(h2, wh_ref[...], (((1,), (1,)), ((), ())),
                         preferred_element_type=jnp.float32) + bh_ref[...][None, :]
    h3 = jnp.maximum(h3, 0.0)
    mu3 = jnp.mean(h3, axis=0, keepdims=True)
    var3 = jnp.mean(h3 * h3, axis=0, keepdims=True) - mu3 * mu3
    h3 = (g3_ref[...][None, :] * (h3 - mu3) / jnp.sqrt(var3 + 1e-5)
          + be3_ref[...][None, :])

    o = lax.dot_general(h3, wf_ref[...], (((1,), (1,)), ((), ())),
                        preferred_element_type=jnp.float32) + bf_ref[...][None, :]
    m = jnp.max(o, axis=1, keepdims=True)
    z = o - m
    lse = jnp.log(jnp.sum(jnp.exp(z), axis=1, keepdims=True))
    out_ref[...] = z - lse


def _dense1(agg, cnt, x, W1l, W1r, b1, g1, be1):
    return pl.pallas_call(
        _dense1_body,
        out_shape=jax.ShapeDtypeStruct((N, 256), jnp.float32),
    )(agg, cnt, x, W1l, W1r, b1, g1, be1)


def _dense2(agg, cnt, h1, W2l, W2r, b2, g2, be2, Wh, bh, g3, be3, Wf, bf):
    return pl.pallas_call(
        _dense2_body,
        out_shape=jax.ShapeDtypeStruct((N, 64), jnp.float32),
    )(agg, cnt, h1, W2l, W2r, b2, g2, be2, Wh, bh, g3, be3, Wf, bf)


def _sample(edge_index):
    src = edge_index[0]
    dst = edge_index[1]
    ones = jnp.ones((E,), dtype=jnp.float32)
    deg_in = jax.ops.segment_sum(ones, dst, N)
    score = deg_in[dst].astype(jnp.int32)
    key = src * (E + 1) + score
    order = jnp.argsort(key)
    s_src = src[order]
    s_dst = dst[order]
    is_start = jnp.concatenate([jnp.array([True]), s_src[1:] != s_src[:-1]])
    idx = jnp.arange(E)
    starts = lax.cummax(jnp.where(is_start, idx, -1))
    rank = idx - starts
    out_deg = jax.ops.segment_sum(ones, src, N)
    gsize = out_deg[s_src]
    w = (rank.astype(jnp.float32) >= (gsize - TOP_K)).astype(jnp.float32)
    return s_src, s_dst, w


def kernel(x, edge_index, batch, W1l, W1r, b1, g1, be1, W2l, W2r, b2, g2, be2,
           Wh, bh, g3, be3, Wf, bf):
    s_src, s_dst, w = _sample(edge_index)

    def agg(feat):
        msg = feat[s_src] * w[:, None]
        ssum = jax.ops.segment_sum(msg, s_dst, N)
        return ssum

    cnt = jax.ops.segment_sum(w, s_dst, N)
    agg1 = agg(x)
    h1 = _dense1(agg1, cnt, x, W1l, W1r, b1, g1, be1)
    agg2 = agg(h1)
    out = _dense2(agg2, cnt, h1, W2l, W2r, b2, g2, be2, Wh, bh, g3, be3, Wf, bf)
    return out


# speedup vs baseline: 2.0864x; 2.0841x over previous
import functools

import jax
import jax.numpy as jnp
from jax import lax
from jax.experimental import pallas as pl
from jax.experimental.pallas import tpu as pltpu
from jax.experimental.pallas import tpu_sc as plsc

N = 10000
E = 160000
TOP_K = 8
NP = 10240
EP = E + 256
NW = 32
SEL = NP * TOP_K

_MESH = plsc.VectorSubcoreMesh(core_axis_name="c", subcore_axis_name="s",
                               num_cores=2, num_subcores=16)
_SC_PARAMS = pltpu.CompilerParams(needs_layout_passes=False)
_IOTA = lambda: lax.iota(jnp.int32, 16)


def _worker_id():
    return lax.axis_index("s") * 2 + lax.axis_index("c")


def _zero_i32(ref, n):
    def body(i, _):
        ref[pl.ds(i * 16, 16)] = jnp.zeros((16,), jnp.int32)
        return 0
    lax.fori_loop(0, n // 16, body, 0)


@functools.partial(
    pl.kernel,
    out_type=jax.ShapeDtypeStruct((2, NP), jnp.int32),
    mesh=_MESH,
    compiler_params=_SC_PARAMS,
    scratch_types=[
        pltpu.VMEM((5008,), jnp.int32),
        pltpu.VMEM((NP,), jnp.int32),
        pltpu.VMEM((640,), jnp.int32),
        pltpu.VMEM((640,), jnp.int32),
        pltpu.VMEM_SHARED((16, NP), jnp.int32),
    ],
)
def _sc_deg(dst_hbm, out_hbm, d_v, hist_v, acc_v, tmp_v, sh):
    c = lax.axis_index("c")
    s = lax.axis_index("s")
    w = _worker_id()
    start = pl.multiple_of((5000 * w) & ~15, 16)
    nxt = (5000 * (w + 1)) & ~15
    nxt = jnp.where(w == NW - 1, E, nxt)
    size = nxt - start
    pltpu.sync_copy(dst_hbm.at[pl.ds(start, 5008)], d_v)
    _zero_i32(hist_v, NP)
    ones = jnp.ones((16,), jnp.int32)
    iota = _IOTA()

    def body(i, _):
        d = d_v[pl.ds(i * 16, 16)]
        valid = (i * 16 + iota) < size
        d = jnp.where(valid, d, NP - 1)
        plsc.addupdate_scatter(hist_v, [d], ones)
        return 0
    lax.fori_loop(0, 313, body, 0)

    pltpu.sync_copy(hist_v, sh.at[s])
    plsc.subcore_barrier()
    _zero_i32(acc_v, 640)
    for t in range(16):
        pltpu.sync_copy(sh.at[t, pl.ds(pl.multiple_of(s * 640, 64), 640)], tmp_v)
        for i in range(40):
            acc_v[pl.ds(i * 16, 16)] += tmp_v[pl.ds(i * 16, 16)]
    pltpu.sync_copy(acc_v, out_hbm.at[c, pl.ds(pl.multiple_of(s * 640, 64), 640)])


_BIG = 0x7FFFFFF0

@functools.partial(
    pl.kernel,
    out_type=[jax.ShapeDtypeStruct((SEL,), jnp.int32),
              jax.ShapeDtypeStruct((2, NP), jnp.int32)],
    mesh=_MESH,
    compiler_params=_SC_PARAMS,
    scratch_types=[
        pltpu.VMEM((NP,), jnp.int32),
        pltpu.VMEM((5008,), jnp.int32),
        pltpu.VMEM((5008,), jnp.int32),
        pltpu.VMEM((5024,), jnp.int32),
        pltpu.VMEM((5024,), jnp.int32),
        pltpu.VMEM((5024,), jnp.int32),
        pltpu.VMEM((320 * 8,), jnp.int32),
        pltpu.VMEM((320 * 8,), jnp.int32),
        pltpu.VMEM((320 * 8,), jnp.int32),
        pltpu.VMEM((320,), jnp.int32),
        pltpu.VMEM((320,), jnp.int32),
        pltpu.VMEM((320,), jnp.int32),
        pltpu.VMEM((320,), jnp.int32),
        pltpu.VMEM((2560,), jnp.int32),
        pltpu.VMEM((NP,), jnp.int32),
        pltpu.VMEM((640,), jnp.int32),
        pltpu.VMEM((640,), jnp.int32),
        pltpu.VMEM_SHARED((16, NP), jnp.int32),
    ],
)
def _sc_select(src_hbm, dst_hbm, deg_hbm, sel_hbm, cnt_hbm,
               deg_v, sv_v, dv_v, cs_v, cd_v, cq_v,
               slotS, slotD, slotQ, minS, minQ, minP, claim_v,
               selout_v, cnt_v, acc_v, tmp_v, sh):
    c = lax.axis_index("c")
    s = lax.axis_index("s")
    w = _worker_id()
    lo = w * 320
    iota = _IOTA()
    lane = iota

    pltpu.sync_copy(deg_hbm.at[0], deg_v)
    pltpu.sync_copy(deg_hbm.at[1], cnt_v)

    def dadd(i, _):
        deg_v[pl.ds(i * 16, 16)] += cnt_v[pl.ds(i * 16, 16)]
        return 0
    lax.fori_loop(0, NP // 16, dadd, 0)

    def sinit(i, _):
        slotS[pl.ds(i * 16, 16)] = jnp.full((16,), -1, jnp.int32)
        slotQ[pl.ds(i * 16, 16)] = jnp.full((16,), -1, jnp.int32)
        return 0
    lax.fori_loop(0, 160, sinit, 0)

    def minit(i, _):
        minS[pl.ds(i * 16, 16)] = jnp.full((16,), -1, jnp.int32)
        minQ[pl.ds(i * 16, 16)] = jnp.full((16,), -1, jnp.int32)
        minP[pl.ds(i * 16, 16)] = jnp.zeros((16,), jnp.int32)
        return 0
    lax.fori_loop(0, 20, minit, 0)

    def window(win, _):
        wstart = pl.multiple_of(win * 5008, 16)
        pltpu.sync_copy(src_hbm.at[pl.ds(wstart, 5008)], sv_v)
        pltpu.sync_copy(dst_hbm.at[pl.ds(wstart, 5008)], dv_v)

        def compact(i, off):
            svec = sv_v[pl.ds(i * 16, 16)]
            m = jnp.logical_and(svec >= lo, svec < lo + 320)
            plsc.store_compressed(cs_v.at[pl.ds(off, 16)], svec - lo, mask=m)
            plsc.store_compressed(cd_v.at[pl.ds(off, 16)],
                                  dv_v[pl.ds(i * 16, 16)], mask=m)
            plsc.store_compressed(cq_v.at[pl.ds(off, 16)],
                                  wstart + i * 16 + iota, mask=m)
            return off + jnp.max(plsc.all_reduce_population_count(m))
        cnt = lax.fori_loop(0, 313, compact, jnp.int32(0))

        def select(i, _):
            m2 = (i * 16 + iota) < cnt
            sl = cs_v[pl.ds(i * 16, 16)]
            d = cd_v[pl.ds(i * 16, 16)]
            q = cq_v[pl.ds(i * 16, 16)]
            sc = plsc.load_gather(deg_v, [jnp.where(m2, d, 0)])

            def cond(st):
                return jnp.max(jnp.where(st[0], 1, 0)) > 0

            def round_(st):
                pend = st[0]
                plsc.store_scatter(claim_v, [sl], lane, mask=pend)
                got = plsc.load_gather(claim_v, [sl], mask=pend)
                winl = jnp.logical_and(pend, got == lane)
                cms = plsc.load_gather(minS, [sl], mask=winl)
                cmq = plsc.load_gather(minQ, [sl], mask=winl)
                cmp_ = plsc.load_gather(minP, [sl], mask=winl)
                ins = jnp.logical_or(sc > cms,
                                     jnp.logical_and(sc == cms, q > cmq))
                rep = jnp.logical_and(winl, ins)
                flat = sl * 8 + cmp_
                plsc.store_scatter(slotS, [flat], sc, mask=rep)
                plsc.store_scatter(slotD, [flat], d, mask=rep)
                plsc.store_scatter(slotQ, [flat], q, mask=rep)
                bs = jnp.full((16,), _BIG, jnp.int32)
                bq = jnp.full((16,), _BIG, jnp.int32)
                bp = jnp.zeros((16,), jnp.int32)
                for j in range(8):
                    sj = plsc.load_gather(slotS, [sl * 8 + j], mask=rep)
                    qj = plsc.load_gather(slotQ, [sl * 8 + j], mask=rep)
                    better = jnp.logical_or(
                        sj < bs, jnp.logical_and(sj == bs, qj < bq))
                    bs = jnp.where(better, sj, bs)
                    bq = jnp.where(better, qj, bq)
                    bp = jnp.where(better, j, bp)
                plsc.store_scatter(minS, [sl], bs, mask=rep)
                plsc.store_scatter(minQ, [sl], bq, mask=rep)
                plsc.store_scatter(minP, [sl], bp, mask=rep)
                return (jnp.logical_and(pend, jnp.logical_not(winl)),)

            lax.while_loop(cond, round_, (m2,))
            return 0

        nv = (cnt + 15) >> 4
        lax.fori_loop(0, nv, select, 0)
        return 0

    lax.fori_loop(0, 32, window, 0)

    _zero_i32(cnt_v, NP)
    ones = jnp.ones((16,), jnp.int32)

    def emit(i, _):
        ss = slotS[pl.ds(i * 16, 16)]
        sd = slotD[pl.ds(i * 16, 16)]
        valid = ss >= 0
        selout_v[pl.ds(i * 16, 16)] = jnp.where(valid, sd, -1)
        plsc.addupdate_scatter(cnt_v, [jnp.where(valid, sd, NP - 1)], ones)
        return 0
    lax.fori_loop(0, 160, emit, 0)
    pltpu.sync_copy(selout_v, sel_hbm.at[pl.ds(pl.multiple_of(lo * 8, 64), 2560)])

    pltpu.sync_copy(cnt_v, sh.at[s])
    plsc.subcore_barrier()
    _zero_i32(acc_v, 640)
    for t in range(16):
        pltpu.sync_copy(sh.at[t, pl.ds(pl.multiple_of(s * 640, 64), 640)], tmp_v)
        for i in range(40):
            acc_v[pl.ds(i * 16, 16)] += tmp_v[pl.ds(i * 16, 16)]
    pltpu.sync_copy(acc_v, cnt_hbm.at[c, pl.ds(pl.multiple_of(s * 640, 64), 640)])


@functools.partial(
    pl.kernel,
    out_type=jax.ShapeDtypeStruct((NP, 256), jnp.float32),
    mesh=_MESH,
    compiler_params=_SC_PARAMS,
    scratch_types=[
        pltpu.VMEM((4096,), jnp.int32),
        pltpu.VMEM((4112,), jnp.int32),
        pltpu.VMEM((4112,), jnp.int32),
        pltpu.VMEM((64,), jnp.int32),
        pltpu.VMEM((64, 256), jnp.float32),
        pltpu.VMEM((320, 256), jnp.float32),
        pltpu.SemaphoreType.DMA,
    ],
)
def _sc_agg(feat_hbm, sel_hbm, out_hbm, wsel_v, cdl_v, csr_v, idx_v,
            rows_v, acc_v, sem):
    w = _worker_id()
    lo = w * 320
    iota = _IOTA()

    def zrow(r, _):
        for cc in range(16):
            acc_v[r, pl.ds(cc * 16, 16)] = jnp.zeros((16,), jnp.float32)
        return 0
    lax.fori_loop(0, 320, zrow, 0)

    def window(win, _):
        pltpu.sync_copy(sel_hbm.at[pl.ds(pl.multiple_of(win * 4096, 64), 4096)], wsel_v)

        def compact(i, off):
            d = wsel_v[pl.ds(i * 16, 16)]
            m = jnp.logical_and(d >= lo, d < lo + 320)
            gid = win * 4096 + i * 16 + iota
            plsc.store_compressed(cdl_v.at[pl.ds(off, 16)], d - lo, mask=m)
            plsc.store_compressed(csr_v.at[pl.ds(off, 16)], gid >> 3, mask=m)
            return off + jnp.max(plsc.all_reduce_population_count(m))
        cnt = lax.fori_loop(0, 256, compact, jnp.int32(0))

        def batch(b, _):
            base = b * 64
            nb = jnp.minimum(cnt - base, 64)
            for j in range(4):
                sv = csr_v[pl.ds(base + j * 16, 16)]
                vld = (base + j * 16 + iota) < cnt
                idx_v[pl.ds(j * 16, 16)] = jnp.where(vld, sv, 0)
            pltpu.async_copy(feat_hbm.at[idx_v], rows_v, sem).wait()

            def accum(e, _):
                dvec = cdl_v[pl.ds(base + e, 16)]
                d = dvec[0]
                for cc in range(16):
                    acc_v[d, pl.ds(cc * 16, 16)] += rows_v[e, pl.ds(cc * 16, 16)]
                return 0
            lax.fori_loop(0, nb, accum, 0)
            return 0

        nbat = (cnt + 63) >> 6
        lax.fori_loop(0, nbat, batch, 0)
        return 0

    lax.fori_loop(0, 20, window, 0)
    pltpu.sync_copy(acc_v, out_hbm.at[pl.ds(pl.multiple_of(lo, 64), 320)])


def _dot_t(a, w):
    return lax.dot_general(a, w, (((1,), (1,)), ((), ())),
                           preferred_element_type=jnp.float32)


def _bn_relu(p, g, b):
    mu = jnp.mean(p, axis=0, keepdims=True)
    var = jnp.mean(p * p, axis=0, keepdims=True) - mu * mu
    h = g[None, :] * (p - mu) / jnp.sqrt(var + 1e-5) + b[None, :]
    return jnp.maximum(h, 0.0)


def _dense1_body(agg_ref, cnt_ref, x_ref, w1l_ref, w1r_ref, b1_ref, g1_ref,
                 be1_ref, h1_ref):
    cnt = (cnt_ref[0, :N] + cnt_ref[1, :N]).astype(jnp.float32)
    mean_agg = agg_ref[:N, :] / jnp.maximum(cnt, 1.0)[:, None]
    p = (_dot_t(mean_agg, w1l_ref[...]) + _dot_t(x_ref[...], w1r_ref[...])
         + b1_ref[...][None, :])
    h1_ref[:N, :] = _bn_relu(p, g1_ref[...], be1_ref[...])
    h1_ref[N:, :] = jnp.zeros((NP - N, 256), jnp.float32)


def _dense2_body(agg_ref, cnt_ref, h1_ref, w2l_ref, w2r_ref, b2_ref, g2_ref,
                 be2_ref, wh_ref, bh_ref, g3_ref, be3_ref, wf_ref, bf_ref,
                 out_ref):
    cnt = (cnt_ref[0, :N] + cnt_ref[1, :N]).astype(jnp.float32)
    mean_agg = agg_ref[:N, :] / jnp.maximum(cnt, 1.0)[:, None]
    p = (_dot_t(mean_agg, w2l_ref[...]) + _dot_t(h1_ref[:N, :], w2r_ref[...])
         + b2_ref[...][None, :])
    h2 = _bn_relu(p, g2_ref[...], be2_ref[...])

    h3 = _dot_t(h2, wh_ref[...]) + bh_ref[...][None, :]
    h3 = jnp.maximum(h3, 0.0)
    mu3 = jnp.mean(h3, axis=0, keepdims=True)
    var3 = jnp.mean(h3 * h3, axis=0, keepdims=True) - mu3 * mu3
    h3 = (g3_ref[...][None, :] * (h3 - mu3) / jnp.sqrt(var3 + 1e-5)
          + be3_ref[...][None, :])

    o = _dot_t(h3, wf_ref[...]) + bf_ref[...][None, :]
    m = jnp.max(o, axis=1, keepdims=True)
    z = o - m
    lse = jnp.log(jnp.sum(jnp.exp(z), axis=1, keepdims=True))
    out_ref[...] = z - lse


def _dense1(agg, cnt, x, W1l, W1r, b1, g1, be1):
    return pl.pallas_call(
        _dense1_body,
        out_shape=jax.ShapeDtypeStruct((NP, 256), jnp.float32),
    )(agg, cnt, x, W1l, W1r, b1, g1, be1)


def _dense2(agg, cnt, h1, W2l, W2r, b2, g2, be2, Wh, bh, g3, be3, Wf, bf):
    return pl.pallas_call(
        _dense2_body,
        out_shape=jax.ShapeDtypeStruct((N, 64), jnp.float32),
    )(agg, cnt, h1, W2l, W2r, b2, g2, be2, Wh, bh, g3, be3, Wf, bf)


def kernel(x, edge_index, batch, W1l, W1r, b1, g1, be1, W2l, W2r, b2, g2, be2,
           Wh, bh, g3, be3, Wf, bf):
    src = jnp.concatenate([edge_index[0], jnp.full((EP - E,), -1, jnp.int32)])
    dst = jnp.concatenate([edge_index[1], jnp.zeros((EP - E,), jnp.int32)])

    deg = _sc_deg(dst)
    sel, cnt = _sc_select(src, dst, deg)

    xpad = jnp.concatenate([x, jnp.zeros((NP - N, 256), jnp.float32)])
    agg1 = _sc_agg(xpad, sel)
    h1pad = _dense1(agg1, cnt, x, W1l, W1r, b1, g1, be1)
    agg2 = _sc_agg(h1pad, sel)
    out = _dense2(agg2, cnt, h1pad, W2l, W2r, b2, g2, be2,
                  Wh, bh, g3, be3, Wf, bf)
    return out


# speedup vs baseline: 2.0946x; 1.0039x over previous
import functools

import jax
import jax.numpy as jnp
from jax import lax
from jax.experimental import pallas as pl
from jax.experimental.pallas import tpu as pltpu
from jax.experimental.pallas import tpu_sc as plsc

N = 10000
E = 160000
TOP_K = 8
NP = 10240
EP = E + 256
NW = 32
SEL = NP * TOP_K

_MESH = plsc.VectorSubcoreMesh(core_axis_name="c", subcore_axis_name="s",
                               num_cores=2, num_subcores=16)
_SC_PARAMS = pltpu.CompilerParams(needs_layout_passes=False)
_IOTA = lambda: lax.iota(jnp.int32, 16)


def _worker_id():
    return lax.axis_index("s") * 2 + lax.axis_index("c")


def _zero_i32(ref, n):
    def body(i, _):
        ref[pl.ds(i * 16, 16)] = jnp.zeros((16,), jnp.int32)
        return 0
    lax.fori_loop(0, n // 16, body, 0)


@functools.partial(
    pl.kernel,
    out_type=jax.ShapeDtypeStruct((2, NP), jnp.int32),
    mesh=_MESH,
    compiler_params=_SC_PARAMS,
    scratch_types=[
        pltpu.VMEM((5008,), jnp.int32),
        pltpu.VMEM((NP,), jnp.int32),
        pltpu.VMEM((640,), jnp.int32),
        pltpu.VMEM((640,), jnp.int32),
        pltpu.VMEM_SHARED((16, NP), jnp.int32),
    ],
)
def _sc_deg(dst_hbm, out_hbm, d_v, hist_v, acc_v, tmp_v, sh):
    c = lax.axis_index("c")
    s = lax.axis_index("s")
    w = _worker_id()
    start = pl.multiple_of((5000 * w) & ~15, 16)
    nxt = (5000 * (w + 1)) & ~15
    nxt = jnp.where(w == NW - 1, E, nxt)
    size = nxt - start
    pltpu.sync_copy(dst_hbm.at[pl.ds(start, 5008)], d_v)
    _zero_i32(hist_v, NP)
    ones = jnp.ones((16,), jnp.int32)
    iota = _IOTA()

    def body(i, _):
        d = d_v[pl.ds(i * 16, 16)]
        valid = (i * 16 + iota) < size
        d = jnp.where(valid, d, NP - 1)
        plsc.addupdate_scatter(hist_v, [d], ones)
        return 0
    lax.fori_loop(0, 313, body, 0)

    pltpu.sync_copy(hist_v, sh.at[s])
    plsc.subcore_barrier()
    _zero_i32(acc_v, 640)
    for t in range(16):
        pltpu.sync_copy(sh.at[t, pl.ds(pl.multiple_of(s * 640, 64), 640)], tmp_v)
        for i in range(40):
            acc_v[pl.ds(i * 16, 16)] += tmp_v[pl.ds(i * 16, 16)]
    pltpu.sync_copy(acc_v, out_hbm.at[c, pl.ds(pl.multiple_of(s * 640, 64), 640)])


_BIG = 0x7FFFFFF0

@functools.partial(
    pl.kernel,
    out_type=[jax.ShapeDtypeStruct((SEL,), jnp.int32),
              jax.ShapeDtypeStruct((2, NP), jnp.int32)],
    mesh=_MESH,
    compiler_params=_SC_PARAMS,
    scratch_types=[
        pltpu.VMEM((NP,), jnp.int32),
        pltpu.VMEM((5008,), jnp.int32),
        pltpu.VMEM((5008,), jnp.int32),
        pltpu.VMEM((5024,), jnp.int32),
        pltpu.VMEM((5024,), jnp.int32),
        pltpu.VMEM((5024,), jnp.int32),
        pltpu.VMEM((320 * 8,), jnp.int32),
        pltpu.VMEM((320 * 8,), jnp.int32),
        pltpu.VMEM((320 * 8,), jnp.int32),
        pltpu.VMEM((320,), jnp.int32),
        pltpu.VMEM((320,), jnp.int32),
        pltpu.VMEM((320,), jnp.int32),
        pltpu.VMEM((320,), jnp.int32),
        pltpu.VMEM((2560,), jnp.int32),
        pltpu.VMEM((NP,), jnp.int32),
        pltpu.VMEM((640,), jnp.int32),
        pltpu.VMEM((640,), jnp.int32),
        pltpu.VMEM_SHARED((16, NP), jnp.int32),
    ],
)
def _sc_select(src_hbm, dst_hbm, deg_hbm, sel_hbm, cnt_hbm,
               deg_v, sv_v, dv_v, cs_v, cd_v, cq_v,
               slotS, slotD, slotQ, minS, minQ, minP, claim_v,
               selout_v, cnt_v, acc_v, tmp_v, sh):
    c = lax.axis_index("c")
    s = lax.axis_index("s")
    w = _worker_id()
    lo = w * 320
    iota = _IOTA()
    lane = iota

    pltpu.sync_copy(deg_hbm.at[0], deg_v)
    pltpu.sync_copy(deg_hbm.at[1], cnt_v)

    def dadd(i, _):
        deg_v[pl.ds(i * 16, 16)] += cnt_v[pl.ds(i * 16, 16)]
        return 0
    lax.fori_loop(0, NP // 16, dadd, 0)

    def sinit(i, _):
        slotS[pl.ds(i * 16, 16)] = jnp.full((16,), -1, jnp.int32)
        slotQ[pl.ds(i * 16, 16)] = jnp.full((16,), -1, jnp.int32)
        return 0
    lax.fori_loop(0, 160, sinit, 0)

    def minit(i, _):
        minS[pl.ds(i * 16, 16)] = jnp.full((16,), -1, jnp.int32)
        minQ[pl.ds(i * 16, 16)] = jnp.full((16,), -1, jnp.int32)
        minP[pl.ds(i * 16, 16)] = jnp.zeros((16,), jnp.int32)
        return 0
    lax.fori_loop(0, 20, minit, 0)

    def window(win, _):
        wstart = pl.multiple_of(win * 5008, 16)
        pltpu.sync_copy(src_hbm.at[pl.ds(wstart, 5008)], sv_v)
        pltpu.sync_copy(dst_hbm.at[pl.ds(wstart, 5008)], dv_v)

        def compact(i, off):
            svec = sv_v[pl.ds(i * 16, 16)]
            m = jnp.logical_and(svec >= lo, svec < lo + 320)
            plsc.store_compressed(cs_v.at[pl.ds(off, 16)], svec - lo, mask=m)
            plsc.store_compressed(cd_v.at[pl.ds(off, 16)],
                                  dv_v[pl.ds(i * 16, 16)], mask=m)
            plsc.store_compressed(cq_v.at[pl.ds(off, 16)],
                                  wstart + i * 16 + iota, mask=m)
            return off + jnp.max(plsc.all_reduce_population_count(m))
        cnt = lax.fori_loop(0, 313, compact, jnp.int32(0))

        def select(i, _):
            m2 = (i * 16 + iota) < cnt
            sl = cs_v[pl.ds(i * 16, 16)]
            d = cd_v[pl.ds(i * 16, 16)]
            q = cq_v[pl.ds(i * 16, 16)]
            sc = plsc.load_gather(deg_v, [jnp.where(m2, d, 0)])

            def cond(st):
                return jnp.max(jnp.where(st[0], 1, 0)) > 0

            def round_(st):
                pend = st[0]
                plsc.store_scatter(claim_v, [sl], lane, mask=pend)
                got = plsc.load_gather(claim_v, [sl], mask=pend)
                winl = jnp.logical_and(pend, got == lane)
                cms = plsc.load_gather(minS, [sl], mask=winl)
                cmq = plsc.load_gather(minQ, [sl], mask=winl)
                cmp_ = plsc.load_gather(minP, [sl], mask=winl)
                ins = jnp.logical_or(sc > cms,
                                     jnp.logical_and(sc == cms, q > cmq))
                rep = jnp.logical_and(winl, ins)
                flat = sl * 8 + cmp_
                plsc.store_scatter(slotS, [flat], sc, mask=rep)
                plsc.store_scatter(slotD, [flat], d, mask=rep)
                plsc.store_scatter(slotQ, [flat], q, mask=rep)
                bs = jnp.full((16,), _BIG, jnp.int32)
                bq = jnp.full((16,), _BIG, jnp.int32)
                bp = jnp.zeros((16,), jnp.int32)
                for j in range(8):
                    sj = plsc.load_gather(slotS, [sl * 8 + j], mask=rep)
                    qj = plsc.load_gather(slotQ, [sl * 8 + j], mask=rep)
                    better = jnp.logical_or(
                        sj < bs, jnp.logical_and(sj == bs, qj < bq))
                    bs = jnp.where(better, sj, bs)
                    bq = jnp.where(better, qj, bq)
                    bp = jnp.where(better, j, bp)
                plsc.store_scatter(minS, [sl], bs, mask=rep)
                plsc.store_scatter(minQ, [sl], bq, mask=rep)
                plsc.store_scatter(minP, [sl], bp, mask=rep)
                return (jnp.logical_and(pend, jnp.logical_not(winl)),)

            lax.while_loop(cond, round_, (m2,))
            return 0

        nv = (cnt + 15) >> 4
        lax.fori_loop(0, nv, select, 0)
        return 0

    lax.fori_loop(0, 32, window, 0)

    _zero_i32(cnt_v, NP)
    ones = jnp.ones((16,), jnp.int32)

    def emit(i, _):
        ss = slotS[pl.ds(i * 16, 16)]
        sd = slotD[pl.ds(i * 16, 16)]
        valid = ss >= 0
        selout_v[pl.ds(i * 16, 16)] = jnp.where(valid, sd, -1)
        plsc.addupdate_scatter(cnt_v, [jnp.where(valid, sd, NP - 1)], ones)
        return 0
    lax.fori_loop(0, 160, emit, 0)
    pltpu.sync_copy(selout_v, sel_hbm.at[pl.ds(pl.multiple_of(lo * 8, 64), 2560)])

    pltpu.sync_copy(cnt_v, sh.at[s])
    plsc.subcore_barrier()
    _zero_i32(acc_v, 640)
    for t in range(16):
        pltpu.sync_copy(sh.at[t, pl.ds(pl.multiple_of(s * 640, 64), 640)], tmp_v)
        for i in range(40):
            acc_v[pl.ds(i * 16, 16)] += tmp_v[pl.ds(i * 16, 16)]
    pltpu.sync_copy(acc_v, cnt_hbm.at[c, pl.ds(pl.multiple_of(s * 640, 64), 640)])


@functools.partial(
    pl.kernel,
    out_type=jax.ShapeDtypeStruct((NP, 256), jnp.float32),
    mesh=_MESH,
    compiler_params=_SC_PARAMS,
    scratch_types=[
        pltpu.VMEM((4096,), jnp.int32),
        pltpu.VMEM((4176,), jnp.int32),
        pltpu.VMEM((4176,), jnp.int32),
        pltpu.VMEM((64, 256), jnp.float32),
        pltpu.VMEM((64, 256), jnp.float32),
        pltpu.VMEM((320, 256), jnp.float32),
        pltpu.SemaphoreType.DMA,
        pltpu.SemaphoreType.DMA,
    ],
)
def _sc_agg(feat_hbm, sel_hbm, out_hbm, wsel_v, cdl_v, csr_v,
            rows0_v, rows1_v, acc_v, sem0, sem1):
    w = _worker_id()
    lo = w * 320
    iota = _IOTA()

    def zrow(r, _):
        for cc in range(16):
            acc_v[r, pl.ds(cc * 16, 16)] = jnp.zeros((16,), jnp.float32)
        return 0
    lax.fori_loop(0, 320, zrow, 0)

    def window(win, _):
        pltpu.sync_copy(sel_hbm.at[pl.ds(pl.multiple_of(win * 4096, 64), 4096)], wsel_v)

        def compact(i, off):
            d = wsel_v[pl.ds(i * 16, 16)]
            m = jnp.logical_and(d >= lo, d < lo + 320)
            gid = win * 4096 + i * 16 + iota
            plsc.store_compressed(cdl_v.at[pl.ds(off, 16)], d - lo, mask=m)
            plsc.store_compressed(csr_v.at[pl.ds(off, 16)], gid >> 3, mask=m)
            return off + jnp.max(plsc.all_reduce_population_count(m))
        cnt = lax.fori_loop(0, 256, compact, jnp.int32(0))

        @pl.when(cnt > 0)
        def _():
            def padtail(j, _):
                pos = pl.multiple_of(((cnt + 15) >> 4) * 16, 16) - 16 + j * 16
                keep = (pos + iota) < cnt
                csr_v[pl.ds(pos, 16)] = jnp.where(keep, csr_v[pl.ds(pos, 16)], 0)
                return 0
            lax.fori_loop(0, 5, padtail, 0)

        def accum(rows_v, base, nb):
            def one(e, _):
                dvec = cdl_v[pl.ds(base + e, 16)]
                d = dvec[0]
                for cc in range(16):
                    acc_v[d, pl.ds(cc * 16, 16)] += rows_v[e, pl.ds(cc * 16, 16)]
                return 0
            lax.fori_loop(0, nb, one, 0)

        nbat = (cnt + 63) >> 6

        def pair(p, _):
            b0 = p * 2
            base0 = pl.multiple_of(b0 * 64, 64)
            idx0 = csr_v.at[pl.ds(base0, 64)]
            pltpu.async_copy(feat_hbm.at[idx0], rows0_v, sem0)
            has1 = (b0 + 1) < nbat
            base1 = pl.multiple_of(base0 + 64, 64)
            idx1 = csr_v.at[pl.ds(base1, 64)]

            @pl.when(has1)
            def _():
                pltpu.async_copy(feat_hbm.at[idx1], rows1_v, sem1)

            pltpu.make_async_copy(feat_hbm.at[idx0], rows0_v, sem0).wait()
            accum(rows0_v, base0, jnp.minimum(cnt - base0, 64))

            @pl.when(has1)
            def _():
                pltpu.make_async_copy(feat_hbm.at[idx1], rows1_v, sem1).wait()
                accum(rows1_v, base1, jnp.minimum(cnt - base1, 64))
            return 0

        lax.fori_loop(0, (nbat + 1) >> 1, pair, 0)
        return 0

    lax.fori_loop(0, 20, window, 0)
    pltpu.sync_copy(acc_v, out_hbm.at[pl.ds(pl.multiple_of(lo, 64), 320)])


def _dot_t(a, w):
    return lax.dot_general(a, w, (((1,), (1,)), ((), ())),
                           preferred_element_type=jnp.float32)


def _bn_relu(p, g, b):
    mu = jnp.mean(p, axis=0, keepdims=True)
    var = jnp.mean(p * p, axis=0, keepdims=True) - mu * mu
    h = g[None, :] * (p - mu) / jnp.sqrt(var + 1e-5) + b[None, :]
    return jnp.maximum(h, 0.0)


def _dense1_body(agg_ref, cnt_ref, x_ref, w1l_ref, w1r_ref, b1_ref, g1_ref,
                 be1_ref, h1_ref):
    cnt = (cnt_ref[0, :N] + cnt_ref[1, :N]).astype(jnp.float32)
    mean_agg = agg_ref[:N, :] / jnp.maximum(cnt, 1.0)[:, None]
    p = (_dot_t(mean_agg, w1l_ref[...]) + _dot_t(x_ref[...], w1r_ref[...])
         + b1_ref[...][None, :])
    h1_ref[:N, :] = _bn_relu(p, g1_ref[...], be1_ref[...])
    h1_ref[N:, :] = jnp.zeros((NP - N, 256), jnp.float32)


def _dense2_body(agg_ref, cnt_ref, h1_ref, w2l_ref, w2r_ref, b2_ref, g2_ref,
                 be2_ref, wh_ref, bh_ref, g3_ref, be3_ref, wf_ref, bf_ref,
                 out_ref):
    cnt = (cnt_ref[0, :N] + cnt_ref[1, :N]).astype(jnp.float32)
    mean_agg = agg_ref[:N, :] / jnp.maximum(cnt, 1.0)[:, None]
    p = (_dot_t(mean_agg, w2l_ref[...]) + _dot_t(h1_ref[:N, :], w2r_ref[...])
         + b2_ref[...][None, :])
    h2 = _bn_relu(p, g2_ref[...], be2_ref[...])

    h3 = _dot_t(h2, wh_ref[...]) + bh_ref[...][None, :]
    h3 = jnp.maximum(h3, 0.0)
    mu3 = jnp.mean(h3, axis=0, keepdims=True)
    var3 = jnp.mean(h3 * h3, axis=0, keepdims=True) - mu3 * mu3
    h3 = (g3_ref[...][None, :] * (h3 - mu3) / jnp.sqrt(var3 + 1e-5)
          + be3_ref[...][None, :])

    o = _dot_t(h3, wf_ref[...]) + bf_ref[...][None, :]
    m = jnp.max(o, axis=1, keepdims=True)
    z = o - m
    lse = jnp.log(jnp.sum(jnp.exp(z), axis=1, keepdims=True))
    out_ref[...] = z - lse


def _dense1(agg, cnt, x, W1l, W1r, b1, g1, be1):
    return pl.pallas_call(
        _dense1_body,
        out_shape=jax.ShapeDtypeStruct((NP, 256), jnp.float32),
    )(agg, cnt, x, W1l, W1r, b1, g1, be1)


def _dense2(agg, cnt, h1, W2l, W2r, b2, g2, be2, Wh, bh, g3, be3, Wf, bf):
    return pl.pallas_call(
        _dense2_body,
        out_shape=jax.ShapeDtypeStruct((N, 64), jnp.float32),
    )(agg, cnt, h1, W2l, W2r, b2, g2, be2, Wh, bh, g3, be3, Wf, bf)


def kernel(x, edge_index, batch, W1l, W1r, b1, g1, be1, W2l, W2r, b2, g2, be2,
           Wh, bh, g3, be3, Wf, bf):
    src = jnp.concatenate([edge_index[0], jnp.full((EP - E,), -1, jnp.int32)])
    dst = jnp.concatenate([edge_index[1], jnp.zeros((EP - E,), jnp.int32)])

    deg = _sc_deg(dst)
    sel, cnt = _sc_select(src, dst, deg)

    xpad = jnp.concatenate([x, jnp.zeros((NP - N, 256), jnp.float32)])
    agg1 = _sc_agg(xpad, sel)
    h1pad = _dense1(agg1, cnt, x, W1l, W1r, b1, g1, be1)
    agg2 = _sc_agg(h1pad, sel)
    out = _dense2(agg2, cnt, h1pad, W2l, W2r, b2, g2, be2,
                  Wh, bh, g3, be3, Wf, bf)
    return out
